# Initial kernel scaffold; baseline (speedup 1.0000x reference)
import jax
import jax.numpy as jnp
from jax.experimental import pallas as pl


def kernel(nodes, ur_history_lists, rating_history_lists, u2e_w, r2e_w, rating2e_w, w_r1_w, w_r1_b, w_r2_w, w_r2_b, att1_w, att1_b, att2_w, att2_b, att3_w, att3_b):
    raise NotImplementedError("write your pallas kernel here")


# speedup vs baseline: 5.2079x; 5.2079x over previous
import functools

import jax
import jax.numpy as jnp
from jax import lax
from jax.experimental import pallas as pl
from jax.experimental.pallas import tpu as pltpu
from jax.experimental.pallas import tpu_sc as plsc

D = 32
L = 50
CHUNK = 128
NC, NS = 2, 16
NW = NC * NS


def _sc_gather(r2e_w, u2e_w, idx3, nodes2):
    nw, n_chunks, chunk = idx3.shape
    _, npw = nodes2.shape
    n_rows = nw * n_chunks * chunk
    b_rows = nw * npw
    per_w = n_chunks * chunk

    mesh = plsc.VectorSubcoreMesh(core_axis_name="c", subcore_axis_name="s")

    @functools.partial(
        pl.kernel,
        mesh=mesh,
        compiler_params=pltpu.CompilerParams(use_tc_tiling_on_sc=False),
        out_type=(
            jax.ShapeDtypeStruct((n_rows, D), jnp.float32),
            jax.ShapeDtypeStruct((b_rows, D), jnp.float32),
        ),
        scratch_types=[
            pltpu.VMEM((n_chunks, chunk), jnp.int32),
            pltpu.VMEM((chunk, D), jnp.float32),
            pltpu.VMEM((chunk, D), jnp.float32),
            pltpu.VMEM((npw,), jnp.int32),
            pltpu.VMEM((npw, D), jnp.float32),
            pltpu.SemaphoreType.DMA,
            pltpu.SemaphoreType.DMA,
            pltpu.SemaphoreType.DMA,
        ],
    )
    def k(r2e_hbm, u2e_hbm, idx_hbm, nodes_hbm, eur_out, urep_out,
          idx_v, rows0, rows1, nidx_v, nrows_v, sem0, sem1, nsem):
        cid = lax.axis_index("c")
        sid = lax.axis_index("s")
        wid = sid * NC + cid
        base = wid * per_w

        pltpu.sync_copy(nodes_hbm.at[wid], nidx_v)
        node_gather = pltpu.make_async_copy(u2e_hbm.at[nidx_v], nrows_v, nsem)
        node_gather.start()

        pltpu.sync_copy(idx_hbm.at[wid], idx_v)

        def start(j, rows, sem):
            pltpu.make_async_copy(r2e_hbm.at[idx_v.at[j]], rows, sem).start()

        def wait_store(j, rows, sem):
            pltpu.make_async_copy(r2e_hbm.at[idx_v.at[j]], rows, sem).wait()
            pltpu.sync_copy(rows, eur_out.at[pl.ds(base + j * chunk, chunk)])

        start(0, rows0, sem0)

        def body(i, carry):
            j0 = 2 * i
            start(j0 + 1, rows1, sem1)
            wait_store(j0, rows0, sem0)

            @pl.when(j0 + 2 < n_chunks)
            def _():
                start(j0 + 2, rows0, sem0)

            wait_store(j0 + 1, rows1, sem1)
            return carry

        lax.fori_loop(0, n_chunks // 2, body, 0)

        node_gather.wait()
        pltpu.sync_copy(nrows_v, urep_out.at[pl.ds(wid * npw, npw)])

    return k(r2e_w, u2e_w, idx3, nodes2)


def _tc_body(eur_ref, rat_ref, urep_ref, r2e_ref, w1_ref, b1_ref, w2_ref,
             b2_ref, a1_ref, a1b_ref, a2_ref, a2b_ref, a3_ref, a3b_ref,
             out_ref, ucon, m_run, d_run, acc):
    f32 = jnp.float32
    l = pl.program_id(0)
    mm = functools.partial(jnp.dot, preferred_element_type=f32)

    w1a = w1_ref[:D, :]
    w1b = w1_ref[D:, :]
    a1a = a1_ref[:D, :]
    a1bw = a1_ref[D:, :]

    @pl.when(l == 0)
    def _():
        ucon[...] = mm(urep_ref[...], a1bw) + a1b_ref[...]
        m_run[...] = jnp.full(m_run.shape, -1e30, f32)
        d_run[...] = jnp.zeros(d_run.shape, f32)
        acc[...] = jnp.zeros(acc.shape, f32)

    rproj = mm(r2e_ref[...], w1b)
    lane8 = lax.broadcasted_iota(jnp.int32, (1, 8), 1)
    oh = (rat_ref[...] == lane8).astype(f32)

    x = eur_ref[...]
    h = jnp.maximum(mm(x, w1a) + mm(oh, rproj) + b1_ref[...], 0.0)
    o = jnp.maximum(mm(h, w2_ref[...]) + b2_ref[...], 0.0)
    a1 = jnp.maximum(mm(o, a1a) + ucon[...], 0.0)
    a2 = jnp.maximum(mm(a1, a2_ref[...]) + a2b_ref[...], 0.0)
    s = mm(a2, a3_ref[...]) + a3b_ref[...]

    m_prev = m_run[...]
    m_new = jnp.maximum(m_prev, s)
    alpha = jnp.exp(m_prev - m_new)
    p = jnp.exp(s - m_new)
    m_run[...] = m_new
    d_new = d_run[...] * alpha + p
    d_run[...] = d_new
    acc_new = acc[...] * alpha + p * o
    acc[...] = acc_new

    @pl.when(l == L - 1)
    def _():
        out_ref[...] = acc_new / d_new


def _tc_attention(eur, rat_flat, urep, r2e_pad, w1, b1, w2, b2, a1w, a1b,
                  a2w, a2b, a3w, a3b):
    b_nodes = urep.shape[0]
    m = b_nodes

    grid = (L,)
    specs = [
        pl.BlockSpec((m, D), lambda l: (l, 0)),
        pl.BlockSpec((m, 1), lambda l: (l, 0)),
        pl.BlockSpec((m, D), lambda l: (0, 0)),
        pl.BlockSpec((8, D), lambda l: (0, 0)),
        pl.BlockSpec((2 * D, D), lambda l: (0, 0)),
        pl.BlockSpec((1, D), lambda l: (0, 0)),
        pl.BlockSpec((D, D), lambda l: (0, 0)),
        pl.BlockSpec((1, D), lambda l: (0, 0)),
        pl.BlockSpec((2 * D, D), lambda l: (0, 0)),
        pl.BlockSpec((1, D), lambda l: (0, 0)),
        pl.BlockSpec((D, D), lambda l: (0, 0)),
        pl.BlockSpec((1, D), lambda l: (0, 0)),
        pl.BlockSpec((D, 1), lambda l: (0, 0)),
        pl.BlockSpec((1, 1), lambda l: (0, 0)),
    ]
    return pl.pallas_call(
        _tc_body,
        grid=grid,
        in_specs=specs,
        out_specs=pl.BlockSpec((m, D), lambda l: (0, 0)),
        out_shape=jax.ShapeDtypeStruct((b_nodes, D), jnp.float32),
        scratch_shapes=[
            pltpu.VMEM((m, D), jnp.float32),
            pltpu.VMEM((m, 1), jnp.float32),
            pltpu.VMEM((m, 1), jnp.float32),
            pltpu.VMEM((m, D), jnp.float32),
        ],
    )(eur, rat_flat, urep, r2e_pad, w1, b1, w2, b2, a1w, a1b, a2w, a2b,
      a3w, a3b)


def kernel(nodes, ur_history_lists, rating_history_lists, u2e_w, r2e_w,
           rating2e_w, w_r1_w, w_r1_b, w_r2_w, w_r2_b, att1_w, att1_b,
           att2_w, att2_b, att3_w, att3_b):
    b_nodes = nodes.shape[0]
    n_rows = b_nodes * L
    per_w = n_rows // NW

    idx3 = (ur_history_lists.astype(jnp.int32).T
            .reshape(NW, per_w // CHUNK, CHUNK))
    nodes2 = nodes.astype(jnp.int32).reshape(NW, b_nodes // NW)

    eur, urep = _sc_gather(r2e_w, u2e_w, idx3, nodes2)

    rat_flat = (rating_history_lists.astype(jnp.int32).T
                .reshape(n_rows, 1))
    r2e_pad = jnp.zeros((8, D), jnp.float32).at[:5].set(rating2e_w)

    return _tc_attention(
        eur, rat_flat, urep, r2e_pad,
        w_r1_w, w_r1_b.reshape(1, D),
        w_r2_w, w_r2_b.reshape(1, D),
        att1_w, att1_b.reshape(1, D),
        att2_w, att2_b.reshape(1, D),
        att3_w, att3_b.reshape(1, 1),
    )
